# baseline (speedup 1.0000x reference)
import jax
import jax.numpy as jnp
from jax.experimental import pallas as pl
from jax.experimental.pallas import tpu as pltpu

U, P, N, D, C, O = 256, 256, 1024, 128, 256, 128
UB = 8


def _pool_body(ub_ref, sxyz_ref, axyz_ref, sf_ref, bf_ref, g1_ref, g2_ref):
    ubb = ub_ref[...]
    cmin = ubb[:, 0:3] - 0.5 * ubb[:, 3:6]
    cmax = ubb[:, 0:3] + 0.5 * ubb[:, 3:6]
    sx = sxyz_ref[0:1, :]
    sy = sxyz_ref[1:2, :]
    sz = sxyz_ref[2:3, :]
    ax = axyz_ref[0:1, :]
    ay = axyz_ref[1:2, :]
    az = axyz_ref[2:3, :]
    sf = sf_ref[...]
    bf = bf_ref[...]
    ms = ((sx >= cmin[:, 0:1]) & (cmax[:, 0:1] >= sx)
          & (sy >= cmin[:, 1:2]) & (cmax[:, 1:2] >= sy)
          & (sz >= cmin[:, 2:3]) & (cmax[:, 2:3] >= sz))
    ma = ((ax >= cmin[:, 0:1]) & (cmax[:, 0:1] >= ax)
          & (ay >= cmin[:, 1:2]) & (cmax[:, 1:2] >= ay)
          & (az >= cmin[:, 2:3]) & (cmax[:, 2:3] >= az))
    for b in range(UB):
        t1 = jnp.where(ms[b:b + 1, :], sf, 0.0)
        g1_ref[b, :, :] = jnp.max(t1, axis=1, keepdims=True).reshape(1, C)
        t2 = jnp.where(ma[b:b + 1, :], bf, 0.0)
        g2_ref[b, :, :] = jnp.max(t2, axis=1, keepdims=True).reshape(1, D)


def _head_body(g1_ref, g2_ref, bfu_ref, w_ref, b_ref, out_ref):
    w = w_ref[...]
    dn = (((0,), (1,)), ((), ()))
    acc = jax.lax.dot_general(g1_ref[...], w[:, :C], dn,
                              preferred_element_type=jnp.float32)
    acc = acc + jax.lax.dot_general(g2_ref[...], w[:, C:C + D], dn,
                                    preferred_element_type=jnp.float32)
    acc = acc + jax.lax.dot_general(bfu_ref[...], w[:, C + D:], dn,
                                    preferred_element_type=jnp.float32)
    a = jnp.abs(acc + b_ref[...] + 1e-6)
    out_ref[...] = a / (1.0 + a)


def kernel(union_box, box_features, agg_xyz, seed_feature, seed_xyz,
           box_feature_union, W, b):
    ub = jnp.pad(union_box[0], ((0, 0), (0, 2)))
    sxyzT = jnp.pad(seed_xyz.T, ((0, 5), (0, 0)))
    axyzT = jnp.pad(agg_xyz.T, ((0, 5), (0, 0)))
    sf = seed_feature.astype(jnp.float16).astype(jnp.float32)
    bfT = box_features.T.astype(jnp.float16).astype(jnp.float32)
    g1, g2 = pl.pallas_call(
        _pool_body,
        grid=(U // UB,),
        in_specs=[
            pl.BlockSpec((UB, 8), lambda u: (u, 0)),
            pl.BlockSpec((8, N), lambda u: (0, 0)),
            pl.BlockSpec((8, P), lambda u: (0, 0)),
            pl.BlockSpec((C, N), lambda u: (0, 0)),
            pl.BlockSpec((D, P), lambda u: (0, 0)),
        ],
        out_specs=[
            pl.BlockSpec((UB, 1, C), lambda u: (u, 0, 0)),
            pl.BlockSpec((UB, 1, D), lambda u: (u, 0, 0)),
        ],
        out_shape=[
            jax.ShapeDtypeStruct((U, 1, C), jnp.float32),
            jax.ShapeDtypeStruct((U, 1, D), jnp.float32),
        ],
    )(ub, sxyzT, axyzT, sf, bfT)
    g1t = g1.reshape(U, C).T
    g2t = g2.reshape(U, D).T
    bfuT = box_feature_union[:, 0, :].T
    out = pl.pallas_call(
        _head_body,
        out_shape=jax.ShapeDtypeStruct((U, O), jnp.float32),
    )(g1t, g2t, bfuT, W, b.reshape(1, O))
    return out
